# baseline (speedup 1.0000x reference)
import functools

import jax
import jax.numpy as jnp
from jax import lax
from jax.experimental import pallas as pl
from jax.experimental.pallas import tpu as pltpu
from jax.experimental.pallas import tpu_sc as plsc

B, D, E, C = 16384, 768, 64, 6
NC, NS = 2, 16
NW = NC * NS
NSLICE = 4
BS = B // NSLICE
TPW = BS // NW
CHUNK = 64
NCH = TPW // CHUNK
NEG = -1e30


def _mlp_body(x_ref, w1_ref, b1_ref, wc_ref, bc_ref, wwe_ref, bwe_ref,
              wew_ref, bew_ref, we_ref, ew_ref, cls_ref):
    x = x_ref[...]
    h = jnp.dot(x, w1_ref[...], preferred_element_type=jnp.float32) + b1_ref[...]
    h = jax.nn.gelu(h)
    we_ref[...] = jnp.dot(h[:, D:2 * D], wwe_ref[...],
                          preferred_element_type=jnp.float32) + bwe_ref[...]
    ew_ref[...] = jnp.dot(h[:, 2 * D:], wew_ref[...],
                          preferred_element_type=jnp.float32) + bew_ref[...]
    cls_ref[...] = jnp.dot(h[:, :D], wc_ref[...],
                           preferred_element_type=jnp.float32) + bc_ref[...]


@jax.jit
def _mlp_run(x, W1, b1, Wc, bc, Wwe, bwe, Wew, bew):
    r = 2048
    grid = BS // r
    full = lambda shape: pl.BlockSpec(shape, lambda i: (0, 0))
    return pl.pallas_call(
        _mlp_body,
        grid=(grid,),
        in_specs=[
            pl.BlockSpec((r, D), lambda i: (i, 0)),
            full((D, 3 * D)),
            full((1, 3 * D)),
            full((D, C * E)),
            full((1, C * E)),
            full((D, E)),
            full((1, E)),
            full((D, E)),
            full((1, E)),
        ],
        out_specs=[
            pl.BlockSpec((r, E), lambda i: (i, 0)),
            pl.BlockSpec((r, E), lambda i: (i, 0)),
            pl.BlockSpec((r, C * E), lambda i: (i, 0)),
        ],
        out_shape=[
            jax.ShapeDtypeStruct((BS, E), jnp.float32),
            jax.ShapeDtypeStruct((BS, E), jnp.float32),
            jax.ShapeDtypeStruct((BS, C * E), jnp.float32),
        ],
    )(x, W1, b1, Wc, bc, Wwe, bwe, Wew, bew)


def _vsort(r):
    return plsc.sort_key_val(r, r)[0]


def _sort64(rows):
    s = [_vsort(r) for r in rows]

    def merge2(a, b):
        rb = jnp.flip(b, 0)
        return _vsort(jnp.minimum(a, rb)), _vsort(jnp.maximum(a, rb))

    l0, h0 = merge2(s[0], s[1])
    l1, h1 = merge2(s[2], s[3])
    x0, x1, x2, x3 = l0, h0, jnp.flip(h1, 0), jnp.flip(l1, 0)
    y0 = jnp.minimum(x0, x2)
    y2 = jnp.maximum(x0, x2)
    y1 = jnp.minimum(x1, x3)
    y3 = jnp.maximum(x1, x3)
    z0 = jnp.minimum(y0, y1)
    z1 = jnp.maximum(y0, y1)
    z2 = jnp.minimum(y2, y3)
    z3 = jnp.maximum(y2, y3)
    return [_vsort(z0), _vsort(z1), _vsort(z2), _vsort(z3)]


def _sc_body(we_hbm, ew_hbm, cls_hbm, n_hbm, out_hbm,
             we_v, ew_v, n_v, srt_v, cls_v, out_v, sem):
    wid = lax.axis_index("s") * NC + lax.axis_index("c")
    base = wid * TPW

    def issue(k, slot):
        b0 = base + k * CHUNK
        return [
            pltpu.async_copy(we_hbm.at[pl.ds(b0, CHUNK)], we_v.at[slot], sem.at[slot, 0]),
            pltpu.async_copy(ew_hbm.at[pl.ds(b0, CHUNK)], ew_v.at[slot], sem.at[slot, 1]),
            pltpu.async_copy(n_hbm.at[pl.ds(b0, CHUNK)], n_v.at[slot], sem.at[slot, 2]),
            pltpu.async_copy(cls_hbm.at[pl.ds(b0, CHUNK)], cls_v.at[slot], sem.at[slot, 3]),
        ]

    cps = [issue(0, 0), issue(1, 1)]
    lane = jnp.arange(16, dtype=jnp.int32)

    for k in range(NCH):
        slot = k & 1
        for cp in cps[slot]:
            cp.wait()

        def one_token(i, p, slot):
            wes = [we_v[slot, i, pl.ds(16 * j, 16)] for j in range(4)]
            srt = _sort64(wes)
            for j in range(4):
                srt_v[p, pl.ds(16 * j, 16)] = srt[j]
            nvec = plsc.load_gather(n_v, [jnp.full((16,), slot, jnp.int32),
                                          jnp.full((16,), i, jnp.int32)])
            n_eff = jnp.where(nvec < 1, E, jnp.minimum(nvec, E))
            tvec = plsc.load_gather(srt_v, [jnp.full((16,), p, jnp.int32),
                                            E - n_eff])
            keeps = [w >= tvec for w in wes]
            ews = [ew_v[slot, i, pl.ds(16 * j, 16)] for j in range(4)]
            mk = [jnp.where(keeps[j], ews[j], NEG) for j in range(4)]
            m = jnp.max(jnp.maximum(jnp.maximum(mk[0], mk[1]),
                                    jnp.maximum(mk[2], mk[3])))
            wv = [jnp.where(keeps[j], jnp.exp(mk[j] - m), 0.0) for j in range(4)]
            wsum = jnp.sum(wv[0] + wv[1] + wv[2] + wv[3])
            cl = [[cls_v[slot, i, pl.ds(c * E + 16 * j, 16)] for c in range(C)]
                  for j in range(4)]
            ex = []
            coef = []
            for j in range(4):
                mj = cl[j][0]
                for c in range(1, C):
                    mj = jnp.maximum(mj, cl[j][c])
                exj = [jnp.exp(cl[j][c] - mj) for c in range(C)]
                zj = exj[0]
                for c in range(1, C):
                    zj = zj + exj[c]
                ex.append(exj)
                coef.append(wv[j] / (zj * wsum))
            outvec = jnp.zeros((16,), jnp.float32)
            for c in range(C):
                num = coef[0] * ex[0][c]
                for j in range(1, 4):
                    num = num + coef[j] * ex[j][c]
                outvec = jnp.where(lane == c, jnp.sum(num), outvec)
            plsc.store_scatter(out_v, [jnp.full((16,), i, jnp.int32), lane],
                               outvec, mask=lane < C)

        def tok_body(i, carry, slot=slot):
            one_token(i, 0, slot)
            return carry

        lax.fori_loop(0, CHUNK, tok_body, 0)
        if k + 2 < NCH:
            cps[slot] = issue(k + 2, slot)
        pltpu.sync_copy(out_v, out_hbm.at[pl.ds(base + k * CHUNK, CHUNK)])


_sc_routing = functools.partial(
    pl.kernel,
    mesh=plsc.VectorSubcoreMesh(core_axis_name="c", subcore_axis_name="s"),
    out_type=jax.ShapeDtypeStruct((BS, C), jnp.float32),
    compiler_params=pltpu.CompilerParams(needs_layout_passes=False),
    scratch_types=[
        pltpu.VMEM((2, CHUNK, E), jnp.float32),
        pltpu.VMEM((2, CHUNK, E), jnp.float32),
        pltpu.VMEM((2, CHUNK), jnp.int32),
        pltpu.VMEM((2, E), jnp.float32),
        pltpu.VMEM((2, CHUNK, C * E), jnp.float32),
        pltpu.VMEM((CHUNK, C), jnp.float32),
        pltpu.SemaphoreType.DMA((2, 4)),
    ],
)(_sc_body)


def kernel(x, n_experts, cls_W1, cls_b1, cls_W2, cls_b2,
           we_W1, we_b1, we_W2, we_b2, ew_W1, ew_b1, ew_W2, ew_b2):
    W1 = jnp.concatenate([cls_W1, we_W1, ew_W1], axis=1)
    b1 = jnp.concatenate([cls_b1, we_b1, ew_b1], axis=0).reshape(1, 3 * D)
    Wc = cls_W2.reshape(D, E, C).transpose(0, 2, 1).reshape(D, C * E)
    bc = cls_b2.reshape(E, C).transpose(1, 0).reshape(1, C * E)
    outs = []
    for s in range(NSLICE):
        we, ew, cls = _mlp_run(x[s * BS:(s + 1) * BS], W1, b1, Wc, bc,
                               we_W2, we_b2.reshape(1, E),
                               ew_W2, ew_b2.reshape(1, E))
        outs.append(_sc_routing(we, ew, cls, n_experts[s * BS:(s + 1) * BS]))
    return jnp.concatenate(outs, axis=0)


# speedup vs baseline: 1.0893x; 1.0893x over previous
import functools

import jax
import jax.numpy as jnp
from jax import lax
from jax.experimental import pallas as pl
from jax.experimental.pallas import tpu as pltpu
from jax.experimental.pallas import tpu_sc as plsc

B, D, E, C = 16384, 768, 64, 6
NC, NS = 2, 16
NW = NC * NS
NSLICE = 4
BS = B // NSLICE
TPW = BS // NW
CHUNK = 64
NCH = TPW // CHUNK
NEG = -1e30


def _mlp_body(x_ref, w1_ref, b1_ref, wc_ref, bc_ref, wwe_ref, bwe_ref,
              wew_ref, bew_ref, we_ref, ew_ref, cls_ref):
    x = x_ref[...]
    h = jnp.dot(x, w1_ref[...], preferred_element_type=jnp.float32) + b1_ref[...]
    h = jax.nn.gelu(h)
    we_ref[...] = jnp.dot(h[:, D:2 * D], wwe_ref[...],
                          preferred_element_type=jnp.float32) + bwe_ref[...]
    ew_ref[...] = jnp.dot(h[:, 2 * D:], wew_ref[...],
                          preferred_element_type=jnp.float32) + bew_ref[...]
    cls_ref[...] = jnp.dot(h[:, :D], wc_ref[...],
                           preferred_element_type=jnp.float32) + bc_ref[...]


@jax.jit
def _mlp_run(x, W1, b1, Wc, bc, Wwe, bwe, Wew, bew):
    r = 1024
    grid = BS // r
    full = lambda shape: pl.BlockSpec(shape, lambda i: (0, 0))
    return pl.pallas_call(
        _mlp_body,
        grid=(grid,),
        in_specs=[
            pl.BlockSpec((r, D), lambda i: (i, 0)),
            full((D, 3 * D)),
            full((1, 3 * D)),
            full((D, C * E)),
            full((1, C * E)),
            full((D, E)),
            full((1, E)),
            full((D, E)),
            full((1, E)),
        ],
        out_specs=[
            pl.BlockSpec((r, E), lambda i: (i, 0)),
            pl.BlockSpec((r, E), lambda i: (i, 0)),
            pl.BlockSpec((r, C * E), lambda i: (i, 0)),
        ],
        out_shape=[
            jax.ShapeDtypeStruct((BS, E), jnp.float32),
            jax.ShapeDtypeStruct((BS, E), jnp.float32),
            jax.ShapeDtypeStruct((BS, C * E), jnp.float32),
        ],
    )(x, W1, b1, Wc, bc, Wwe, bwe, Wew, bew)


def _vsort(r):
    return plsc.sort_key_val(r, r)[0]


def _sort64(rows):
    s = [_vsort(r) for r in rows]

    def merge2(a, b):
        rb = jnp.flip(b, 0)
        return _vsort(jnp.minimum(a, rb)), _vsort(jnp.maximum(a, rb))

    l0, h0 = merge2(s[0], s[1])
    l1, h1 = merge2(s[2], s[3])
    x0, x1, x2, x3 = l0, h0, jnp.flip(h1, 0), jnp.flip(l1, 0)
    y0 = jnp.minimum(x0, x2)
    y2 = jnp.maximum(x0, x2)
    y1 = jnp.minimum(x1, x3)
    y3 = jnp.maximum(x1, x3)
    z0 = jnp.minimum(y0, y1)
    z1 = jnp.maximum(y0, y1)
    z2 = jnp.minimum(y2, y3)
    z3 = jnp.maximum(y2, y3)
    return [_vsort(z0), _vsort(z1), _vsort(z2), _vsort(z3)]


def _sc_body(we_hbm, ew_hbm, cls_hbm, n_hbm, out_hbm,
             we_v, ew_v, n_v, srt_v, cls_v, out_v, sem):
    wid = lax.axis_index("s") * NC + lax.axis_index("c")
    base = wid * TPW

    def issue(k, slot):
        b0 = base + k * CHUNK
        return [
            pltpu.async_copy(we_hbm.at[pl.ds(b0, CHUNK)], we_v.at[slot], sem.at[slot, 0]),
            pltpu.async_copy(ew_hbm.at[pl.ds(b0, CHUNK)], ew_v.at[slot], sem.at[slot, 1]),
            pltpu.async_copy(n_hbm.at[pl.ds(b0, CHUNK)], n_v.at[slot], sem.at[slot, 2]),
            pltpu.async_copy(cls_hbm.at[pl.ds(b0, CHUNK)], cls_v.at[slot], sem.at[slot, 3]),
        ]

    cps = [issue(0, 0), issue(1, 1)]
    lane = jnp.arange(16, dtype=jnp.int32)

    for k in range(NCH):
        slot = k & 1
        for cp in cps[slot]:
            cp.wait()

        def one_token(i, p, slot):
            wes = [we_v[slot, i, pl.ds(16 * j, 16)] for j in range(4)]
            srt = _sort64(wes)
            for j in range(4):
                srt_v[p, pl.ds(16 * j, 16)] = srt[j]
            nvec = plsc.load_gather(n_v, [jnp.full((16,), slot, jnp.int32),
                                          jnp.full((16,), i, jnp.int32)])
            n_eff = jnp.where(nvec < 1, E, jnp.minimum(nvec, E))
            tvec = plsc.load_gather(srt_v, [jnp.full((16,), p, jnp.int32),
                                            E - n_eff])
            keeps = [w >= tvec for w in wes]
            ews = [ew_v[slot, i, pl.ds(16 * j, 16)] for j in range(4)]
            mk = [jnp.where(keeps[j], ews[j], NEG) for j in range(4)]
            m = jnp.max(jnp.maximum(jnp.maximum(mk[0], mk[1]),
                                    jnp.maximum(mk[2], mk[3])))
            wv = [jnp.where(keeps[j], jnp.exp(mk[j] - m), 0.0) for j in range(4)]
            wsum = jnp.sum(wv[0] + wv[1] + wv[2] + wv[3])
            cl = [[cls_v[slot, i, pl.ds(c * E + 16 * j, 16)] for c in range(C)]
                  for j in range(4)]
            ex = []
            coef = []
            for j in range(4):
                mj = cl[j][0]
                for c in range(1, C):
                    mj = jnp.maximum(mj, cl[j][c])
                exj = [jnp.exp(cl[j][c] - mj) for c in range(C)]
                zj = exj[0]
                for c in range(1, C):
                    zj = zj + exj[c]
                ex.append(exj)
                coef.append(wv[j] / (zj * wsum))
            outvec = jnp.zeros((16,), jnp.float32)
            for c in range(C):
                num = coef[0] * ex[0][c]
                for j in range(1, 4):
                    num = num + coef[j] * ex[j][c]
                outvec = jnp.where(lane == c, jnp.sum(num), outvec)
            plsc.store_scatter(out_v, [jnp.full((16,), i, jnp.int32), lane],
                               outvec, mask=lane < C)

        def tok_body(i, carry, slot=slot):
            one_token(i, 0, slot)
            return carry

        lax.fori_loop(0, CHUNK, tok_body, 0)
        if k + 2 < NCH:
            cps[slot] = issue(k + 2, slot)
        pltpu.sync_copy(out_v, out_hbm.at[pl.ds(base + k * CHUNK, CHUNK)])


_sc_routing = functools.partial(
    pl.kernel,
    mesh=plsc.VectorSubcoreMesh(core_axis_name="c", subcore_axis_name="s"),
    out_type=jax.ShapeDtypeStruct((BS, C), jnp.float32),
    compiler_params=pltpu.CompilerParams(needs_layout_passes=False),
    scratch_types=[
        pltpu.VMEM((2, CHUNK, E), jnp.float32),
        pltpu.VMEM((2, CHUNK, E), jnp.float32),
        pltpu.VMEM((2, CHUNK), jnp.int32),
        pltpu.VMEM((2, E), jnp.float32),
        pltpu.VMEM((2, CHUNK, C * E), jnp.float32),
        pltpu.VMEM((CHUNK, C), jnp.float32),
        pltpu.SemaphoreType.DMA((2, 4)),
    ],
)(_sc_body)


def kernel(x, n_experts, cls_W1, cls_b1, cls_W2, cls_b2,
           we_W1, we_b1, we_W2, we_b2, ew_W1, ew_b1, ew_W2, ew_b2):
    W1 = jnp.concatenate([cls_W1, we_W1, ew_W1], axis=1)
    b1 = jnp.concatenate([cls_b1, we_b1, ew_b1], axis=0).reshape(1, 3 * D)
    Wc = cls_W2.reshape(D, E, C).transpose(0, 2, 1).reshape(D, C * E)
    bc = cls_b2.reshape(E, C).transpose(1, 0).reshape(1, C * E)
    outs = []
    for s in range(NSLICE):
        we, ew, cls = _mlp_run(x[s * BS:(s + 1) * BS], W1, b1, Wc, bc,
                               we_W2, we_b2.reshape(1, E),
                               ew_W2, ew_b2.reshape(1, E))
        outs.append(_sc_routing(we, ew, cls, n_experts[s * BS:(s + 1) * BS]))
    return jnp.concatenate(outs, axis=0)
